# baseline (speedup 1.0000x reference)
import functools

import jax
import jax.numpy as jnp
from jax.experimental import pallas as pl
from jax.experimental.pallas import tpu as pltpu


def _prep_chunk(x_ref, w_ref, kp_ref, mp_ref, qp_ref, vp_ref, slot, *, tc, n):
    tc_dim, bblk, d = x_ref.shape
    xb = x_ref[...].reshape(tc * bblk, d)
    proj = jax.lax.dot_general(xb, w_ref[...], (((1,), (1,)), ((), ())),
                               preferred_element_type=jnp.float32)
    proj = proj.reshape(tc, bblk, 4 * n)
    k_all = proj[:, :, :n]
    m_all = proj[:, :, 3 * n:]
    k_all = k_all / (jnp.sqrt(jnp.sum(k_all * k_all, axis=-1, keepdims=True)) + 1e-6)
    m_all = m_all / (jnp.sqrt(jnp.sum(m_all * m_all, axis=-1, keepdims=True)) + 1e-6)
    zpad = jnp.zeros_like(k_all)
    kp_ref[slot] = jnp.concatenate([k_all, zpad], axis=-1)
    mp_ref[slot] = jnp.concatenate([m_all, zpad], axis=-1)
    q_all = proj[:, :, 2 * n:3 * n]
    qp_ref[slot] = jnp.concatenate([q_all, q_all], axis=-1)
    vp_ref[slot] = proj[:, :, n:2 * n][:, :, :, None]


def _gate_cell_kernel(x_ref, xn_ref, s0_ref, m0_ref, w_ref, bs_ref, bm_ref,
                      out_ref, s_out_ref, m_out_ref,
                      s_ref, m_ref, kp_ref, mp_ref, qp_ref, vp_ref,
                      *, tc, n, nt):
    t_idx = pl.program_id(1)
    slot = jax.lax.rem(t_idx, 2)

    @pl.when(t_idx == 0)
    def _():
        zero = jnp.zeros_like(s0_ref[...])
        s_ref[...] = jnp.concatenate([s0_ref[...], zero], axis=-1)
        m_ref[...] = jnp.concatenate([m0_ref[...], zero], axis=-1)
        _prep_chunk(x_ref, w_ref, kp_ref, mp_ref, qp_ref, vp_ref, 0,
                    tc=tc, n=n)

    bs = bs_ref[...]
    bs = jnp.concatenate([bs, bs], axis=-1)[None]
    bm = bm_ref[...]
    bm = jnp.concatenate([bm, bm], axis=-1)[None]

    def sig(z):
        return 0.5 * jnp.tanh(0.5 * z) + 0.5

    for t in range(tc):
        kb = kp_ref[slot, t][:, None, :]
        mb = mp_ref[slot, t][:, None, :]
        qb = qp_ref[slot, t][:, None, :]
        v_r = vp_ref[slot, t]
        S = s_ref[...]
        M = m_ref[...]
        M_k = jnp.sum(M * kb, axis=2, keepdims=True)
        G_S = sig(M + M_k * kb + bs)
        s_delta = v_r - jnp.sum(S * kb, axis=2, keepdims=True)
        S = G_S * S + s_delta * kb
        s_ref[...] = S
        S_m = jnp.sum(S * mb, axis=2, keepdims=True)
        G_M = sig(S + S_m * mb + bm)
        m_delta = s_delta - jnp.sum(M * mb, axis=2, keepdims=True)
        M = G_M * M + m_delta * mb
        m_ref[...] = M
        Sq = jnp.sum(S * qb, axis=2, keepdims=True)
        out_ref[t] = Sq * Sq * sig(Sq)

    @pl.when(t_idx < nt - 1)
    def _():
        _prep_chunk(xn_ref, w_ref, kp_ref, mp_ref, qp_ref, vp_ref, 1 - slot,
                    tc=tc, n=n)

    @pl.when(t_idx == nt - 1)
    def _():
        s_out_ref[...] = s_ref[:, :, :n]
        m_out_ref[...] = m_ref[:, :, :n]


def kernel(x, S0, M0, W_kvqm, B_S, B_M):
    T, B, D = x.shape
    N = B_S.shape[0]
    BBLK = 32 if B % 32 == 0 else B
    TC = 8 if T % 8 == 0 else 1
    nb = B // BBLK
    nt = T // TC

    body = functools.partial(_gate_cell_kernel, tc=TC, n=N, nt=nt)

    out, S, M = pl.pallas_call(
        body,
        grid=(nb, nt),
        in_specs=[
            pl.BlockSpec((TC, BBLK, D), lambda b, t: (t, b, 0)),
            pl.BlockSpec((TC, BBLK, D),
                         lambda b, t: (jnp.minimum(t + 1, nt - 1), b, 0)),
            pl.BlockSpec((BBLK, N, N), lambda b, t: (b, 0, 0)),
            pl.BlockSpec((BBLK, N, N), lambda b, t: (b, 0, 0)),
            pl.BlockSpec((4 * N, D), lambda b, t: (0, 0)),
            pl.BlockSpec((N, N), lambda b, t: (0, 0)),
            pl.BlockSpec((N, N), lambda b, t: (0, 0)),
        ],
        out_specs=[
            pl.BlockSpec((TC, BBLK, N, 1), lambda b, t: (t, b, 0, 0)),
            pl.BlockSpec((BBLK, N, N), lambda b, t: (b, 0, 0)),
            pl.BlockSpec((BBLK, N, N), lambda b, t: (b, 0, 0)),
        ],
        out_shape=[
            jax.ShapeDtypeStruct((T, B, N, 1), jnp.float32),
            jax.ShapeDtypeStruct((B, N, N), jnp.float32),
            jax.ShapeDtypeStruct((B, N, N), jnp.float32),
        ],
        scratch_shapes=[
            pltpu.VMEM((BBLK, N, 2 * N), jnp.float32),
            pltpu.VMEM((BBLK, N, 2 * N), jnp.float32),
            pltpu.VMEM((2, TC, BBLK, 2 * N), jnp.float32),
            pltpu.VMEM((2, TC, BBLK, 2 * N), jnp.float32),
            pltpu.VMEM((2, TC, BBLK, 2 * N), jnp.float32),
            pltpu.VMEM((2, TC, BBLK, N, 1), jnp.float32),
        ],
        compiler_params=pltpu.CompilerParams(
            dimension_semantics=("parallel", "arbitrary"),
            vmem_limit_bytes=56 * 1024 * 1024,
        ),
    )(x, x, S0, M0, W_kvqm, B_S, B_M)
    return out.reshape(T, B, N), S, M


# speedup vs baseline: 1.2177x; 1.2177x over previous
import functools

import jax
import jax.numpy as jnp
from jax.experimental import pallas as pl
from jax.experimental.pallas import tpu as pltpu


def _gate_cell_kernel(x_ref, s0_ref, m0_ref, w_ref, bs_ref, bm_ref,
                      out_ref, s_out_ref, m_out_ref, s_ref, m_ref,
                      *, tc, n, nt):
    t_idx = pl.program_id(1)

    @pl.when(t_idx == 0)
    def _():
        zero = jnp.zeros_like(s0_ref[...])
        s_ref[...] = jnp.concatenate([s0_ref[...], zero], axis=-1)
        m_ref[...] = jnp.concatenate([m0_ref[...], zero], axis=-1)

    tc_dim, bblk, d = x_ref.shape
    xb = x_ref[...].reshape(tc * bblk, d)
    proj = jax.lax.dot_general(xb, w_ref[...], (((1,), (1,)), ((), ())),
                               preferred_element_type=jnp.float32)
    proj = proj.reshape(tc, bblk, 4 * n)

    bs = bs_ref[...]
    bs = jnp.concatenate([bs, bs], axis=-1)[None]
    bm = bm_ref[...]
    bm = jnp.concatenate([bm, bm], axis=-1)[None]

    k_all = proj[:, :, :n]
    m_all = proj[:, :, 3 * n:]
    k_all = k_all / (jnp.sqrt(jnp.sum(k_all * k_all, axis=-1, keepdims=True)) + 1e-6)
    m_all = m_all / (jnp.sqrt(jnp.sum(m_all * m_all, axis=-1, keepdims=True)) + 1e-6)
    zpad = jnp.zeros_like(k_all)
    k_all = jnp.concatenate([k_all, zpad], axis=-1)
    m_all = jnp.concatenate([m_all, zpad], axis=-1)
    q_all = proj[:, :, 2 * n:3 * n]
    q_all = jnp.concatenate([q_all, q_all], axis=-1)
    v_all = proj[:, :, n:2 * n][:, :, :, None]

    def sig(z):
        return 0.5 * jnp.tanh(0.5 * z) + 0.5

    for t in range(tc):
        kb = k_all[t][:, None, :]
        mb = m_all[t][:, None, :]
        qb = q_all[t][:, None, :]
        v_r = v_all[t]
        S = s_ref[...]
        M = m_ref[...]
        M_k = jnp.sum(M * kb, axis=2, keepdims=True)
        G_S = sig(M + M_k * kb + bs)
        s_delta = v_r - jnp.sum(S * kb, axis=2, keepdims=True)
        S = G_S * S + s_delta * kb
        s_ref[...] = S
        S_m = jnp.sum(S * mb, axis=2, keepdims=True)
        G_M = sig(S + S_m * mb + bm)
        m_delta = s_delta - jnp.sum(M * mb, axis=2, keepdims=True)
        M = G_M * M + m_delta * mb
        m_ref[...] = M
        Sq = jnp.sum(S * qb, axis=2, keepdims=True)
        out_ref[t] = Sq * Sq * sig(Sq)

    @pl.when(t_idx == nt - 1)
    def _():
        s_out_ref[...] = s_ref[:, :, :n]
        m_out_ref[...] = m_ref[:, :, :n]


def kernel(x, S0, M0, W_kvqm, B_S, B_M):
    T, B, D = x.shape
    N = B_S.shape[0]
    BBLK = 32 if B % 32 == 0 else B
    TC = 8 if T % 8 == 0 else 1
    nb = B // BBLK
    nt = T // TC

    body = functools.partial(_gate_cell_kernel, tc=TC, n=N, nt=nt)

    out, S, M = pl.pallas_call(
        body,
        grid=(nb, nt),
        in_specs=[
            pl.BlockSpec((TC, BBLK, D), lambda b, t: (t, b, 0)),
            pl.BlockSpec((BBLK, N, N), lambda b, t: (b, 0, 0)),
            pl.BlockSpec((BBLK, N, N), lambda b, t: (b, 0, 0)),
            pl.BlockSpec((4 * N, D), lambda b, t: (0, 0)),
            pl.BlockSpec((N, N), lambda b, t: (0, 0)),
            pl.BlockSpec((N, N), lambda b, t: (0, 0)),
        ],
        out_specs=[
            pl.BlockSpec((TC, BBLK, N, 1), lambda b, t: (t, b, 0, 0)),
            pl.BlockSpec((BBLK, N, N), lambda b, t: (b, 0, 0)),
            pl.BlockSpec((BBLK, N, N), lambda b, t: (b, 0, 0)),
        ],
        out_shape=[
            jax.ShapeDtypeStruct((T, B, N, 1), jnp.float32),
            jax.ShapeDtypeStruct((B, N, N), jnp.float32),
            jax.ShapeDtypeStruct((B, N, N), jnp.float32),
        ],
        scratch_shapes=[
            pltpu.VMEM((BBLK, N, 2 * N), jnp.float32),
            pltpu.VMEM((BBLK, N, 2 * N), jnp.float32),
        ],
        compiler_params=pltpu.CompilerParams(
            dimension_semantics=("parallel", "arbitrary"),
            vmem_limit_bytes=56 * 1024 * 1024,
        ),
    )(x, S0, M0, W_kvqm, B_S, B_M)
    return out.reshape(T, B, N), S, M
